# Initial kernel scaffold; baseline (speedup 1.0000x reference)
import jax
import jax.numpy as jnp
from jax.experimental import pallas as pl


def kernel(hidden_states, topk_indices, topk_weights, W_gate, W_up, W_down):
    raise NotImplementedError("write your pallas kernel here")


# speedup vs baseline: 7.7533x; 7.7533x over previous
import functools

import jax
import jax.numpy as jnp
from jax import lax
from jax.experimental import pallas as pl
from jax.experimental.pallas import tpu as pltpu
from jax.experimental.pallas import tpu_sc as plsc

T = 4096
D = 768
F = 2048
E = 16
K = 2

B = 256
NB = (T * K) // B + (E - 1)
NPAD = NB * B

NC = 2
NS = 16
NW = NC * NS
TOK_W = T // NW
SUB = 64

@functools.cache
def _mesh():
    return plsc.VectorSubcoreMesh(
        core_axis_name="c", subcore_axis_name="s", num_cores=NC, num_subcores=NS
    )


def _routing(topk_indices):
    flat = topk_indices.reshape(-1).astype(jnp.int32)
    onehot = (flat[:, None] == jnp.arange(E, dtype=jnp.int32)[None, :]).astype(
        jnp.int32
    )
    csum = jnp.cumsum(onehot, axis=0)
    rank = jnp.take_along_axis(csum, flat[:, None], axis=1)[:, 0] - 1
    counts = csum[-1]
    nblk = (counts + B - 1) // B
    blk_off = jnp.concatenate(
        [jnp.zeros((1,), jnp.int32), jnp.cumsum(nblk)[:-1].astype(jnp.int32)]
    )
    dst = blk_off[flat] * B + rank
    num_used = jnp.sum(nblk).astype(jnp.int32)
    bids = jnp.arange(NB, dtype=jnp.int32)
    be = jnp.sum((bids[:, None] >= blk_off[None, :]).astype(jnp.int32), axis=1) - 1
    meta = jnp.concatenate([num_used[None], be.astype(jnp.int32)])
    dst2 = dst.reshape(T, K)
    return dst2[:, 0], dst2[:, 1], meta


@functools.cache
def _dispatch_kernel():
    @functools.partial(
        pl.kernel,
        out_type=jax.ShapeDtypeStruct((NPAD, D), jnp.float32),
        mesh=_mesh(),
        scratch_types=[
            pltpu.VMEM((TOK_W, D), jnp.float32),
            pltpu.VMEM((TOK_W,), jnp.int32),
            pltpu.VMEM((TOK_W,), jnp.int32),
            pltpu.SemaphoreType.DMA,
            pltpu.SemaphoreType.DMA,
        ],
    )
    def _dispatch(hid_hbm, dst0_hbm, dst1_hbm, xpad_hbm, rows_v, i0_v, i1_v, s0, s1):
        wid = lax.axis_index("s") * NC + lax.axis_index("c")
        base = wid * TOK_W
        pltpu.sync_copy(hid_hbm.at[pl.ds(base, TOK_W)], rows_v)
        pltpu.sync_copy(dst0_hbm.at[pl.ds(base, TOK_W)], i0_v)
        pltpu.sync_copy(dst1_hbm.at[pl.ds(base, TOK_W)], i1_v)
        c0 = pltpu.async_copy(rows_v, xpad_hbm.at[i0_v], s0)
        c1 = pltpu.async_copy(rows_v, xpad_hbm.at[i1_v], s1)
        c0.wait()
        c1.wait()

    return _dispatch


@functools.cache
def _combine_kernel():
    @functools.partial(
        pl.kernel,
        out_type=jax.ShapeDtypeStruct((T, D), jnp.float32),
        mesh=_mesh(),
        scratch_types=[
            pltpu.VMEM((SUB, D), jnp.float32),
            pltpu.VMEM((SUB, D), jnp.float32),
            pltpu.VMEM((SUB,), jnp.int32),
            pltpu.VMEM((SUB,), jnp.int32),
            pltpu.VMEM((SUB + 16,), jnp.float32),
            pltpu.VMEM((SUB + 16,), jnp.float32),
            pltpu.SemaphoreType.DMA,
            pltpu.SemaphoreType.DMA,
        ],
    )
    def _combine(
        ypad_hbm, dst0_hbm, dst1_hbm, w0_hbm, w1_hbm, out_hbm,
        y0_v, y1_v, i0_v, i1_v, w0_v, w1_v, s0, s1,
    ):
        wid = lax.axis_index("s") * NC + lax.axis_index("c")
        for sub in range(TOK_W // SUB):
            base = wid * TOK_W + sub * SUB
            pltpu.sync_copy(dst0_hbm.at[pl.ds(base, SUB)], i0_v)
            pltpu.sync_copy(dst1_hbm.at[pl.ds(base, SUB)], i1_v)
            pltpu.sync_copy(w0_hbm.at[pl.ds(base, SUB)], w0_v.at[pl.ds(0, SUB)])
            pltpu.sync_copy(w1_hbm.at[pl.ds(base, SUB)], w1_v.at[pl.ds(0, SUB)])
            pltpu.async_copy(ypad_hbm.at[i0_v], y0_v, s0).wait()
            pltpu.async_copy(ypad_hbm.at[i1_v], y1_v, s1).wait()

            def row_body(r, carry):
                a = w0_v[pl.ds(r, 16)][0]
                b = w1_v[pl.ds(r, 16)][0]
                for cb in range(D // 16):
                    sl = pl.ds(cb * 16, 16)
                    y0_v[r, sl] = a * y0_v[r, sl] + b * y1_v[r, sl]
                return carry

            lax.fori_loop(0, SUB, row_body, 0)
            pltpu.sync_copy(y0_v, out_hbm.at[pl.ds(base, SUB)])

    return _combine


def _ffn_body(meta_ref, x_ref, wg_ref, wu_ref, wd_ref, y_ref):
    b = pl.program_id(0)

    @pl.when(b < meta_ref[0])
    def _():
        x = x_ref[...]
        g = jnp.dot(x, wg_ref[0], preferred_element_type=jnp.float32)
        u = jnp.dot(x, wu_ref[0], preferred_element_type=jnp.float32)
        h = g * jax.nn.sigmoid(g) * u
        y_ref[...] = jnp.dot(h, wd_ref[0], preferred_element_type=jnp.float32)


def _ffn(meta, x_pad, W_gate, W_up, W_down):
    grid_spec = pltpu.PrefetchScalarGridSpec(
        num_scalar_prefetch=1,
        grid=(NB,),
        in_specs=[
            pl.BlockSpec((B, D), lambda b, m: (b, 0)),
            pl.BlockSpec((1, D, F), lambda b, m: (m[1 + b], 0, 0)),
            pl.BlockSpec((1, D, F), lambda b, m: (m[1 + b], 0, 0)),
            pl.BlockSpec((1, F, D), lambda b, m: (m[1 + b], 0, 0)),
        ],
        out_specs=pl.BlockSpec((B, D), lambda b, m: (b, 0)),
    )
    return pl.pallas_call(
        _ffn_body,
        grid_spec=grid_spec,
        out_shape=jax.ShapeDtypeStruct((NPAD, D), jnp.float32),
        compiler_params=pltpu.CompilerParams(
            dimension_semantics=("arbitrary",),
        ),
    )(meta, x_pad, W_gate, W_up, W_down)


def kernel(hidden_states, topk_indices, topk_weights, W_gate, W_up, W_down):
    dst0, dst1, meta = _routing(topk_indices)
    x_pad = _dispatch_kernel()(hidden_states, dst0, dst1)
    y_pad = _ffn(meta, x_pad, W_gate, W_up, W_down)
    w = topk_weights.astype(jnp.float32)
    out = _combine_kernel()(y_pad, dst0, dst1, w[:, 0], w[:, 1])
    return out
